# baseline (speedup 1.0000x reference)
import jax
import jax.numpy as jnp
from jax.experimental import pallas as pl
from jax.experimental.pallas import tpu as pltpu

_C = 80
_K = 300
_KP = 384
_S = 98
_ROWS = _S * 128
_NQ = 20000
_CHUNK = 2000


def _postproc_kernel(x_ref, bt_ref, sz_ref, scores_ref, labels_ref, bx_ref,
                     xs_ref):
    xs_ref[...] = x_ref[0]
    m1 = jnp.max(xs_ref[...], axis=2)
    sj_iota = (jax.lax.broadcasted_iota(jnp.int32, (_S, 128), 0) * 128
               + jax.lax.broadcasted_iota(jnp.int32, (_S, 128), 1))
    lane_iota = jax.lax.broadcasted_iota(jnp.int32, (1, 128), 1)
    k_iota = jax.lax.broadcasted_iota(jnp.int32, (1, _KP), 1)
    big = jnp.int32(2 ** 30)

    def body(i, carry):
        m1, vals, idxs = carry
        g = jnp.max(m1)
        r = jnp.min(jnp.where(m1 == g, sj_iota, big))
        s = r // 128
        j = r % 128
        row = xs_ref[pl.ds(s, 1), pl.ds(j, 1), :]
        row = row.reshape(1, 128)
        l = jnp.min(jnp.where(row == g, lane_iota, big))
        flat = r * 128 + l
        row2 = jnp.where(lane_iota == l, -jnp.inf, row)
        xs_ref[pl.ds(s, 1), pl.ds(j, 1), :] = row2.reshape(1, 1, 128)
        m1 = jnp.where(sj_iota == r, jnp.max(row2), m1)
        vals = jnp.where(k_iota == i, g, vals)
        idxs = jnp.where(k_iota == i, flat, idxs)
        return m1, vals, idxs

    init = (m1, jnp.full((1, _KP), -jnp.inf, jnp.float32),
            jnp.zeros((1, _KP), jnp.int32))
    _, vals, idxs = jax.lax.fori_loop(0, _K, body, init)

    scores_ref[0, 0:1, :] = jax.nn.sigmoid(vals)
    labels_ref[0, 0:1, :] = idxs % _C
    q = idxs // _C

    acc = jnp.zeros((4, _KP), jnp.float32)
    for s0 in range(0, _NQ, _CHUNK):
        n_iota = s0 + jax.lax.broadcasted_iota(jnp.int32, (_CHUNK, 1), 0)
        oh = (n_iota == q).astype(jnp.float32)
        chunk = bt_ref[0, :, s0:s0 + _CHUNK]
        acc = acc + jax.lax.dot(chunk, oh,
                                preferred_element_type=jnp.float32)

    cx, cy, w, h = (acc[0:1, :], acc[1:2, :], acc[2:3, :], acc[3:4, :])
    b = pl.program_id(0)
    sw = sz_ref[b, 0].astype(jnp.float32)
    sh = sz_ref[b, 1].astype(jnp.float32)
    bx_ref[0, 0:4, :] = jnp.concatenate(
        [(cx - 0.5 * w) * sw, (cy - 0.5 * h) * sh,
         (cx + 0.5 * w) * sw, (cy + 0.5 * h) * sh], axis=0)


def kernel(pred_logits, pred_boxes, orig_target_sizes):
    B, N, C = pred_logits.shape
    flat = pred_logits.reshape(B, N * C)
    pad = _ROWS * 128 - N * C
    flatp = jnp.pad(flat, ((0, 0), (0, pad)), constant_values=-jnp.inf)
    x3 = flatp.reshape(B, _S, 128, 128)
    bt = pred_boxes.transpose(0, 2, 1)

    scores, labels, bx = pl.pallas_call(
        _postproc_kernel,
        grid=(B,),
        in_specs=[
            pl.BlockSpec((1, _S, 128, 128), lambda b: (b, 0, 0, 0)),
            pl.BlockSpec((1, 4, _NQ), lambda b: (b, 0, 0)),
            pl.BlockSpec(memory_space=pltpu.SMEM),
        ],
        out_specs=[
            pl.BlockSpec((1, 8, _KP), lambda b: (b, 0, 0)),
            pl.BlockSpec((1, 8, _KP), lambda b: (b, 0, 0)),
            pl.BlockSpec((1, 8, _KP), lambda b: (b, 0, 0)),
        ],
        out_shape=[
            jax.ShapeDtypeStruct((B, 8, _KP), jnp.float32),
            jax.ShapeDtypeStruct((B, 8, _KP), jnp.int32),
            jax.ShapeDtypeStruct((B, 8, _KP), jnp.float32),
        ],
        scratch_shapes=[pltpu.VMEM((_S, 128, 128), jnp.float32)],
        compiler_params=pltpu.CompilerParams(
            dimension_semantics=("parallel",)),
    )(x3, bt, orig_target_sizes)

    return (scores[:, 0, :_K], labels[:, 0, :_K],
            bx[:, 0:4, :_K].transpose(0, 2, 1))


# speedup vs baseline: 1.0462x; 1.0462x over previous
import jax
import jax.numpy as jnp
from jax.experimental import pallas as pl
from jax.experimental.pallas import tpu as pltpu

_C = 80
_K = 300
_KP = 384
_S = 98
_ROWS = _S * 128
_NQ = 20000
_CHUNK = 2000
_BB = 2


def _postproc_kernel(x_ref, bt_ref, sz_ref, scores_ref, labels_ref, bx_ref,
                     xs_ref):
    xs_ref[...] = x_ref[...]
    sj_iota = (jax.lax.broadcasted_iota(jnp.int32, (_S, 128), 0) * 128
               + jax.lax.broadcasted_iota(jnp.int32, (_S, 128), 1))
    lane_iota = jax.lax.broadcasted_iota(jnp.int32, (1, 128), 1)
    k_iota = jax.lax.broadcasted_iota(jnp.int32, (1, _KP), 1)
    big = jnp.int32(2 ** 30)

    def extract(k, i, m1, vals, idxs):
        g = jnp.max(m1)
        r = jnp.min(jnp.where(m1 == g, sj_iota, big))
        s = r // 128
        j = r % 128
        row = xs_ref[k, pl.ds(s, 1), pl.ds(j, 1), :]
        row = row.reshape(1, 128)
        l = jnp.min(jnp.where(row == g, lane_iota, big))
        flat = r * 128 + l
        row2 = jnp.where(lane_iota == l, -jnp.inf, row)
        xs_ref[k, pl.ds(s, 1), pl.ds(j, 1), :] = row2.reshape(1, 1, 128)
        m1 = jnp.where(sj_iota == r, jnp.max(row2), m1)
        vals = jnp.where(k_iota == i, g, vals)
        idxs = jnp.where(k_iota == i, flat, idxs)
        return m1, vals, idxs

    def body(i, carry):
        m1s, valss, idxss = carry
        outs = [extract(k, i, m1s[k], valss[k], idxss[k]) for k in range(_BB)]
        return (tuple(o[0] for o in outs), tuple(o[1] for o in outs),
                tuple(o[2] for o in outs))

    init = (tuple(jnp.max(xs_ref[k], axis=2) for k in range(_BB)),
            tuple(jnp.full((1, _KP), -jnp.inf, jnp.float32)
                  for _ in range(_BB)),
            tuple(jnp.zeros((1, _KP), jnp.int32) for _ in range(_BB)))
    _, valss, idxss = jax.lax.fori_loop(0, _K, body, init)

    b0 = pl.program_id(0) * _BB
    for k in range(_BB):
        vals, idxs = valss[k], idxss[k]
        scores_ref[k, 0:1, :] = jax.nn.sigmoid(vals)
        labels_ref[k, 0:1, :] = idxs % _C
        q = idxs // _C

        acc = jnp.zeros((4, _KP), jnp.float32)
        for s0 in range(0, _NQ, _CHUNK):
            n_iota = s0 + jax.lax.broadcasted_iota(jnp.int32, (_CHUNK, 1), 0)
            oh = (n_iota == q).astype(jnp.float32)
            chunk = bt_ref[k, :, s0:s0 + _CHUNK]
            acc = acc + jax.lax.dot(chunk, oh,
                                    preferred_element_type=jnp.float32)

        cx, cy, w, h = (acc[0:1, :], acc[1:2, :], acc[2:3, :], acc[3:4, :])
        sw = sz_ref[b0 + k, 0].astype(jnp.float32)
        sh = sz_ref[b0 + k, 1].astype(jnp.float32)
        bx_ref[k, 0:4, :] = jnp.concatenate(
            [(cx - 0.5 * w) * sw, (cy - 0.5 * h) * sh,
             (cx + 0.5 * w) * sw, (cy + 0.5 * h) * sh], axis=0)


def kernel(pred_logits, pred_boxes, orig_target_sizes):
    B, N, C = pred_logits.shape
    flat = pred_logits.reshape(B, N * C)
    pad = _ROWS * 128 - N * C
    flatp = jnp.pad(flat, ((0, 0), (0, pad)), constant_values=-jnp.inf)
    x3 = flatp.reshape(B, _S, 128, 128)
    bt = pred_boxes.transpose(0, 2, 1)

    scores, labels, bx = pl.pallas_call(
        _postproc_kernel,
        grid=(B // _BB,),
        in_specs=[
            pl.BlockSpec((_BB, _S, 128, 128), lambda b: (b, 0, 0, 0)),
            pl.BlockSpec((_BB, 4, _NQ), lambda b: (b, 0, 0)),
            pl.BlockSpec(memory_space=pltpu.SMEM),
        ],
        out_specs=[
            pl.BlockSpec((_BB, 8, _KP), lambda b: (b, 0, 0)),
            pl.BlockSpec((_BB, 8, _KP), lambda b: (b, 0, 0)),
            pl.BlockSpec((_BB, 8, _KP), lambda b: (b, 0, 0)),
        ],
        out_shape=[
            jax.ShapeDtypeStruct((B, 8, _KP), jnp.float32),
            jax.ShapeDtypeStruct((B, 8, _KP), jnp.int32),
            jax.ShapeDtypeStruct((B, 8, _KP), jnp.float32),
        ],
        scratch_shapes=[pltpu.VMEM((_BB, _S, 128, 128), jnp.float32)],
        compiler_params=pltpu.CompilerParams(
            dimension_semantics=("parallel",)),
    )(x3, bt, orig_target_sizes)

    return (scores[:, 0, :_K], labels[:, 0, :_K],
            bx[:, 0:4, :_K].transpose(0, 2, 1))


# speedup vs baseline: 1.0463x; 1.0001x over previous
import jax
import jax.numpy as jnp
from jax.experimental import pallas as pl
from jax.experimental.pallas import tpu as pltpu

_C = 80
_K = 300
_KP = 384
_S = 98
_ROWS = _S * 128
_NQ = 20000
_CHUNK = 2000
_BB = 2


def _postproc_kernel(x_ref, bt_ref, sz_ref, scores_ref, labels_ref, bx_ref):
    xs_ref = x_ref
    sj_iota = (jax.lax.broadcasted_iota(jnp.int32, (_S, 128), 0) * 128
               + jax.lax.broadcasted_iota(jnp.int32, (_S, 128), 1))
    lane_iota = jax.lax.broadcasted_iota(jnp.int32, (1, 128), 1)
    k_iota = jax.lax.broadcasted_iota(jnp.int32, (1, _KP), 1)
    big = jnp.int32(2 ** 30)

    def extract(k, i, m1, vals, idxs):
        g = jnp.max(m1)
        r = jnp.min(jnp.where(m1 == g, sj_iota, big))
        s = r // 128
        j = r % 128
        row = xs_ref[k, pl.ds(s, 1), pl.ds(j, 1), :]
        row = row.reshape(1, 128)
        l = jnp.min(jnp.where(row == g, lane_iota, big))
        flat = r * 128 + l
        row2 = jnp.where(lane_iota == l, -jnp.inf, row)
        xs_ref[k, pl.ds(s, 1), pl.ds(j, 1), :] = row2.reshape(1, 1, 128)
        m1 = jnp.where(sj_iota == r, jnp.max(row2), m1)
        vals = jnp.where(k_iota == i, g, vals)
        idxs = jnp.where(k_iota == i, flat, idxs)
        return m1, vals, idxs

    def body(i, carry):
        m1s, valss, idxss = carry
        outs = [extract(k, i, m1s[k], valss[k], idxss[k]) for k in range(_BB)]
        return (tuple(o[0] for o in outs), tuple(o[1] for o in outs),
                tuple(o[2] for o in outs))

    init = (tuple(jnp.max(xs_ref[k], axis=2) for k in range(_BB)),
            tuple(jnp.full((1, _KP), -jnp.inf, jnp.float32)
                  for _ in range(_BB)),
            tuple(jnp.zeros((1, _KP), jnp.int32) for _ in range(_BB)))
    _, valss, idxss = jax.lax.fori_loop(0, _K, body, init)

    b0 = pl.program_id(0) * _BB
    for k in range(_BB):
        vals, idxs = valss[k], idxss[k]
        scores_ref[k, 0:1, :] = jax.nn.sigmoid(vals)
        labels_ref[k, 0:1, :] = idxs % _C
        q = idxs // _C

        acc = jnp.zeros((4, _KP), jnp.float32)
        for s0 in range(0, _NQ, _CHUNK):
            n_iota = s0 + jax.lax.broadcasted_iota(jnp.int32, (_CHUNK, 1), 0)
            oh = (n_iota == q).astype(jnp.float32)
            chunk = bt_ref[k, :, s0:s0 + _CHUNK]
            acc = acc + jax.lax.dot(chunk, oh,
                                    preferred_element_type=jnp.float32)

        cx, cy, w, h = (acc[0:1, :], acc[1:2, :], acc[2:3, :], acc[3:4, :])
        sw = sz_ref[b0 + k, 0].astype(jnp.float32)
        sh = sz_ref[b0 + k, 1].astype(jnp.float32)
        bx_ref[k, 0:4, :] = jnp.concatenate(
            [(cx - 0.5 * w) * sw, (cy - 0.5 * h) * sh,
             (cx + 0.5 * w) * sw, (cy + 0.5 * h) * sh], axis=0)


def kernel(pred_logits, pred_boxes, orig_target_sizes):
    B, N, C = pred_logits.shape
    flat = pred_logits.reshape(B, N * C)
    pad = _ROWS * 128 - N * C
    flatp = jnp.pad(flat, ((0, 0), (0, pad)), constant_values=-jnp.inf)
    x3 = flatp.reshape(B, _S, 128, 128)
    bt = pred_boxes.transpose(0, 2, 1)

    scores, labels, bx = pl.pallas_call(
        _postproc_kernel,
        grid=(B // _BB,),
        in_specs=[
            pl.BlockSpec((_BB, _S, 128, 128), lambda b: (b, 0, 0, 0)),
            pl.BlockSpec((_BB, 4, _NQ), lambda b: (b, 0, 0)),
            pl.BlockSpec(memory_space=pltpu.SMEM),
        ],
        out_specs=[
            pl.BlockSpec((_BB, 8, _KP), lambda b: (b, 0, 0)),
            pl.BlockSpec((_BB, 8, _KP), lambda b: (b, 0, 0)),
            pl.BlockSpec((_BB, 8, _KP), lambda b: (b, 0, 0)),
        ],
        out_shape=[
            jax.ShapeDtypeStruct((B, 8, _KP), jnp.float32),
            jax.ShapeDtypeStruct((B, 8, _KP), jnp.int32),
            jax.ShapeDtypeStruct((B, 8, _KP), jnp.float32),
        ],
        compiler_params=pltpu.CompilerParams(
            dimension_semantics=("parallel",)),
    )(x3, bt, orig_target_sizes)

    return (scores[:, 0, :_K], labels[:, 0, :_K],
            bx[:, 0:4, :_K].transpose(0, 2, 1))
